# Initial kernel scaffold; baseline (speedup 1.0000x reference)
import jax
import jax.numpy as jnp
from jax.experimental import pallas as pl


def kernel(x, edge_index, batch, eps1, W1a, b1a, W1b, b1b, g1, be1, eps2, W2a, b2a, W2b, b2b, g2, be2, eps3, W3a, b3a, W3b, b3b, g3, be3, lin1_W, lin1_b, lin2_W, lin2_b):
    raise NotImplementedError("write your pallas kernel here")


# speedup vs baseline: 3.3452x; 3.3452x over previous
import functools

import jax
import jax.numpy as jnp
from jax import lax
from jax.experimental import pallas as pl
from jax.experimental.pallas import tpu as pltpu
from jax.experimental.pallas import tpu_sc as plsc

NC = 2
NS = 16
KE = 128
TILE_BLK = NS * KE
CHUNK = 8960
NCHUNK = 6
NPAD = NCHUNK * CHUNK
ACC_R = 8976
ZROWS = 187
WROWS = 112
RBLK = 2000


def _segsum_sc(h, src_arr, dstl_arr, a_arr):
  mesh = plsc.VectorSubcoreMesh(
      core_axis_name="c", subcore_axis_name="s", num_cores=NC, num_subcores=NS)

  @functools.partial(
      pl.kernel,
      out_type=jax.ShapeDtypeStruct((NPAD, 128), jnp.float32),
      mesh=mesh,
      scratch_types=[
          pltpu.VMEM((16,), jnp.int32),
          pltpu.VMEM((KE,), jnp.int32),
          pltpu.VMEM((KE,), jnp.int32),
          pltpu.VMEM((KE, 128), jnp.float32),
          pltpu.VMEM((ZROWS, 128), jnp.float32),
          pltpu.VMEM_SHARED((ACC_R, 128), jnp.float32),
          pltpu.SemaphoreType.DMA,
      ],
  )
  def k(x_hbm, src_hbm, dstl_hbm, a_hbm, out_hbm,
        a_v, src_v, dst_v, rows_v, zb_v, acc, sem):
    core = lax.axis_index("c")
    sub = lax.axis_index("s")
    pltpu.sync_copy(a_hbm, a_v)
    av = a_v[...]

    def geta(i):
      return jnp.squeeze(lax.slice(av, (i,), (i + 1,)))

    zvec = jnp.zeros((16,), jnp.float32)

    def zbody(t, _):
      r = t // 8
      c8 = t % 8
      zb_v[r, pl.ds(c8 * 16, 16)] = zvec
      return 0

    lax.fori_loop(0, ZROWS * 8, zbody, 0)

    for kk in range(NCHUNK // 2):
      cid = core + 2 * kk
      off = jnp.where(core == 0, geta(2 * kk), geta(2 * kk + 1))
      nblk = jnp.where(core == 0, geta(NCHUNK + 2 * kk),
                       geta(NCHUNK + 2 * kk + 1))
      pcnt = nblk * KE

      for s3 in range(3):
        pltpu.sync_copy(zb_v, acc.at[pl.ds(sub * 561 + s3 * ZROWS, ZROWS)])
      plsc.subcore_barrier()

      tstart = pl.multiple_of(off + sub * pcnt, KE)

      def ebody(b, _):
        bs = pl.multiple_of(tstart + b * KE, KE)
        pltpu.sync_copy(src_hbm.at[pl.ds(bs, KE)], src_v)
        pltpu.sync_copy(dstl_hbm.at[pl.ds(bs, KE)], dst_v)
        pltpu.async_copy(x_hbm.at[src_v], rows_v, sem).wait()
        pltpu.sync_copy(rows_v, acc.at[dst_v], add=True)
        return 0

      lax.fori_loop(0, nblk, ebody, 0)
      plsc.subcore_barrier()

      for s7 in range(5):
        lrow = sub * 560 + s7 * WROWS
        grow = cid * CHUNK + lrow
        pltpu.sync_copy(acc.at[pl.ds(lrow, WROWS)], rows_v.at[pl.ds(0, WROWS)])
        pltpu.sync_copy(rows_v.at[pl.ds(0, WROWS)], out_hbm.at[pl.ds(grow, WROWS)])
      plsc.subcore_barrier()

  return k(h, src_arr, dstl_arr, a_arr)


def _mlp_body(epsb_ref, x_ref, agg_ref, w1_ref, b1_ref, w2_ref, b2_ref,
              y_ref, st_ref):
  i = pl.program_id(0)
  h0 = epsb_ref[...] * x_ref[...] + agg_ref[...]
  z = jnp.maximum(
      jnp.dot(h0, w1_ref[...], preferred_element_type=jnp.float32) + b1_ref[...],
      0.0)
  y = jnp.maximum(
      jnp.dot(z, w2_ref[...], preferred_element_type=jnp.float32) + b2_ref[...],
      0.0)
  y_ref[...] = y

  @pl.when(i == 0)
  def _():
    st_ref[...] = jnp.zeros_like(st_ref)

  st_ref[0:1, :] += jnp.sum(y, axis=0, keepdims=True)
  st_ref[1:2, :] += jnp.sum(y * y, axis=0, keepdims=True)


def _mlp(epsb, x, agg, w1, b1, w2, b2, nblocks):
  return pl.pallas_call(
      _mlp_body,
      grid=(nblocks,),
      in_specs=[
          pl.BlockSpec((1, 128), lambda i: (0, 0)),
          pl.BlockSpec((RBLK, 128), lambda i: (i, 0)),
          pl.BlockSpec((RBLK, 128), lambda i: (i, 0)),
          pl.BlockSpec((128, 128), lambda i: (0, 0)),
          pl.BlockSpec((1, 128), lambda i: (0, 0)),
          pl.BlockSpec((128, 128), lambda i: (0, 0)),
          pl.BlockSpec((1, 128), lambda i: (0, 0)),
      ],
      out_specs=[
          pl.BlockSpec((RBLK, 128), lambda i: (i, 0)),
          pl.BlockSpec((8, 128), lambda i: (0, 0)),
      ],
      out_shape=[
          jax.ShapeDtypeStruct((nblocks * RBLK, 128), jnp.float32),
          jax.ShapeDtypeStruct((8, 128), jnp.float32),
      ],
  )(epsb, x, agg, w1, b1, w2, b2)


def _bn_body(n_ref, st_ref, g_ref, be_ref, y_ref, o_ref):
  n = n_ref[0, 0]
  mean = st_ref[0:1, :] / n
  var = st_ref[1:2, :] / n - mean * mean
  scale = g_ref[...] * lax.rsqrt(var + 1e-5)
  o_ref[...] = (y_ref[...] - mean) * scale + be_ref[...]


def _bn(nval, st, g, be, y, nblocks):
  return pl.pallas_call(
      _bn_body,
      grid=(nblocks,),
      in_specs=[
          pl.BlockSpec((1, 1), lambda i: (0, 0)),
          pl.BlockSpec((8, 128), lambda i: (0, 0)),
          pl.BlockSpec((1, 128), lambda i: (0, 0)),
          pl.BlockSpec((1, 128), lambda i: (0, 0)),
          pl.BlockSpec((RBLK, 128), lambda i: (i, 0)),
      ],
      out_specs=pl.BlockSpec((RBLK, 128), lambda i: (i, 0)),
      out_shape=jax.ShapeDtypeStruct((nblocks * RBLK, 128), jnp.float32),
  )(nval, st, g, be, y)


def _bnpool_body(n_ref, st_ref, g_ref, be_ref, y_ref, batch_ref,
                 sums_ref, cnts_ref):
  i = pl.program_id(0)
  n = n_ref[0, 0]
  mean = st_ref[0:1, :] / n
  var = st_ref[1:2, :] / n - mean * mean
  scale = g_ref[...] * lax.rsqrt(var + 1e-5)
  ybn = (y_ref[...] - mean) * scale + be_ref[...]
  b = batch_ref[0, 0, :]
  oh = (lax.broadcasted_iota(jnp.int32, (RBLK, 128), 1) == b[:, None]
        ).astype(jnp.float32)

  @pl.when(i == 0)
  def _():
    sums_ref[...] = jnp.zeros_like(sums_ref)
    cnts_ref[...] = jnp.zeros_like(cnts_ref)

  sums_ref[...] += lax.dot_general(
      oh, ybn, (((0,), (0,)), ((), ())), preferred_element_type=jnp.float32)
  cnts_ref[...] += lax.dot_general(
      oh, jnp.ones((RBLK, 128), jnp.float32), (((0,), (0,)), ((), ())),
      preferred_element_type=jnp.float32)


def _bnpool(nval, st, g, be, y, batch3, nblocks):
  return pl.pallas_call(
      _bnpool_body,
      grid=(nblocks,),
      in_specs=[
          pl.BlockSpec((1, 1), lambda i: (0, 0)),
          pl.BlockSpec((8, 128), lambda i: (0, 0)),
          pl.BlockSpec((1, 128), lambda i: (0, 0)),
          pl.BlockSpec((1, 128), lambda i: (0, 0)),
          pl.BlockSpec((RBLK, 128), lambda i: (i, 0)),
          pl.BlockSpec((1, 1, RBLK), lambda i: (i, 0, 0)),
      ],
      out_specs=[
          pl.BlockSpec((128, 128), lambda i: (0, 0)),
          pl.BlockSpec((128, 128), lambda i: (0, 0)),
      ],
      out_shape=[
          jax.ShapeDtypeStruct((128, 128), jnp.float32),
          jax.ShapeDtypeStruct((128, 128), jnp.float32),
      ],
  )(nval, st, g, be, y, batch3)


def _head_body(sums_ref, cnts_ref, w1_ref, b1_ref, w2_ref, b2_ref, o_ref):
  xr = sums_ref[...] / jnp.maximum(cnts_ref[...], 1.0)
  o = jnp.maximum(
      jnp.dot(xr, w1_ref[...], preferred_element_type=jnp.float32) + b1_ref[...],
      0.0)
  o2 = jnp.dot(o, w2_ref[...], preferred_element_type=jnp.float32) + b2_ref[...]
  m = jnp.max(o2, axis=1, keepdims=True)
  e = jnp.exp(o2 - m)
  s = jnp.sum(e, axis=1, keepdims=True)
  o_ref[...] = (o2 - m) - jnp.log(s)


def _head(sums, cnts, w1, b1, w2, b2, C):
  return pl.pallas_call(
      _head_body,
      out_shape=jax.ShapeDtypeStruct((128, C), jnp.float32),
  )(sums, cnts, w1, b1, w2, b2)


def kernel(x, edge_index, batch, eps1, W1a, b1a, W1b, b1b, g1, be1,
           eps2, W2a, b2a, W2b, b2b, g2, be2,
           eps3, W3a, b3a, W3b, b3b, g3, be3,
           lin1_W, lin1_b, lin2_W, lin2_b):
  N, FIN = x.shape
  H = W1a.shape[1]
  C = lin2_W.shape[1]
  E = edge_index.shape[1]
  nblocks = N // RBLK
  E_pad = E + NCHUNK * TILE_BLK

  src = edge_index[0]
  dst = edge_index[1]
  perm = jnp.argsort(dst)
  src_s = src[perm]
  dst_s = dst[perm]
  bounds = jnp.searchsorted(
      dst_s, jnp.arange(0, NPAD + 1, CHUNK, dtype=jnp.int32)).astype(jnp.int32)
  cnts = jnp.diff(bounds)
  plen = ((cnts + TILE_BLK - 1) // TILE_BLK) * TILE_BLK
  poff = jnp.concatenate(
      [jnp.zeros((1,), jnp.int32), jnp.cumsum(plen)]).astype(jnp.int32)
  p = jnp.arange(E_pad, dtype=jnp.int32)
  seg = jnp.clip(jnp.searchsorted(poff, p, side="right").astype(jnp.int32) - 1,
                 0, NCHUNK - 1)
  l = p - poff[seg]
  valid = l < cnts[seg]
  j = jnp.where(valid, bounds[seg] + l, 0)
  src_arr = jnp.where(valid, src_s[j], 0).astype(jnp.int32)
  dstl_arr = jnp.where(valid, dst_s[j] - seg * CHUNK, CHUNK).astype(jnp.int32)
  a_arr = jnp.concatenate(
      [poff[:NCHUNK], plen // TILE_BLK,
       jnp.zeros((16 - 2 * NCHUNK,), jnp.int32)]).astype(jnp.int32)

  xp = jnp.pad(x, ((0, 0), (0, H - FIN)))
  W1a_p = jnp.pad(W1a, ((0, H - FIN), (0, 0)))
  nv = jnp.full((1, 1), float(N), jnp.float32)

  def layer(h, epsv, wa, ba, wb, bb):
    agg = _segsum_sc(h, src_arr, dstl_arr, a_arr)
    epsb = jnp.full((1, 128), 1.0, jnp.float32) + epsv
    y, st = _mlp(epsb, h, agg, wa, ba.reshape(1, -1), wb, bb.reshape(1, -1),
                 nblocks)
    return y, st

  y1, st1 = layer(xp, eps1, W1a_p, b1a, W1b, b1b)
  h1 = _bn(nv, st1, g1.reshape(1, -1), be1.reshape(1, -1), y1, nblocks)
  y2, st2 = layer(h1, eps2, W2a, b2a, W2b, b2b)
  h2 = _bn(nv, st2, g2.reshape(1, -1), be2.reshape(1, -1), y2, nblocks)
  y3, st3 = layer(h2, eps3, W3a, b3a, W3b, b3b)
  batch3 = batch.reshape(nblocks, 1, RBLK).astype(jnp.int32)
  sums, cnts_pool = _bnpool(nv, st3, g3.reshape(1, -1), be3.reshape(1, -1),
                            y3, batch3, nblocks)
  return _head(sums, cnts_pool, lin1_W, lin1_b.reshape(1, -1),
               lin2_W, lin2_b.reshape(1, -1), C)


# speedup vs baseline: 4.0451x; 1.2092x over previous
import functools

import jax
import jax.numpy as jnp
from jax import lax
from jax.experimental import pallas as pl
from jax.experimental.pallas import tpu as pltpu
from jax.experimental.pallas import tpu_sc as plsc

NC = 2
NS = 16
KE = 128
TILE_BLK = NS * KE
CHUNK = 8960
NCHUNK = 6
NPAD = NCHUNK * CHUNK
ACC_R = 9120
WROWS = 112
RBLK = 2000


def _segsum_sc(h, src_arr, dstl_arr, a_arr):
  mesh = plsc.VectorSubcoreMesh(
      core_axis_name="c", subcore_axis_name="s", num_cores=NC, num_subcores=NS)

  @functools.partial(
      pl.kernel,
      out_type=jax.ShapeDtypeStruct((NPAD, 128), jnp.float32),
      mesh=mesh,
      scratch_types=[
          pltpu.VMEM((16,), jnp.int32),
          pltpu.VMEM((2, KE), jnp.int32),
          pltpu.VMEM((2, KE), jnp.int32),
          pltpu.VMEM((2, KE, 128), jnp.float32),
          pltpu.VMEM((WROWS, 128), jnp.float32),
          pltpu.VMEM_SHARED((ACC_R, 128), jnp.float32),
          pltpu.SemaphoreType.DMA,
          pltpu.SemaphoreType.DMA,
      ],
  )
  def k(x_hbm, src_hbm, dstl_hbm, a_hbm, out_hbm,
        a_v, src2, dst2, rows2, wb_v, acc, sem0, sem1):
    core = lax.axis_index("c")
    sub = lax.axis_index("s")
    pltpu.sync_copy(a_hbm, a_v)
    av = a_v[...]

    def geta(i):
      return jnp.squeeze(lax.slice(av, (i,), (i + 1,)))

    zvec = jnp.zeros((16,), jnp.float32)

    def zbody(t, _):
      zb = t // 8
      c8 = t % 8
      wb_v[zb, pl.ds(c8 * 16, 16)] = zvec
      return 0

    for kk in range(NCHUNK // 2):
      cid = core + 2 * kk
      off = jnp.where(core == 0, geta(2 * kk), geta(2 * kk + 1))
      nblk = jnp.where(core == 0, geta(NCHUNK + 2 * kk),
                       geta(NCHUNK + 2 * kk + 1))
      pcnt = nblk * KE

      lax.fori_loop(0, WROWS * 8, zbody, 0)
      for s6 in range(6):
        pltpu.sync_copy(wb_v, acc.at[pl.ds(sub * 561 + s6 * WROWS, WROWS)])
      plsc.subcore_barrier()

      tstart = pl.multiple_of(off + sub * pcnt, KE)
      sems = (sem0, sem1)

      def issue(b, slot):
        bs = pl.multiple_of(tstart + b * KE, KE)
        pltpu.sync_copy(src_hbm.at[pl.ds(bs, KE)], src2.at[slot])
        pltpu.sync_copy(dstl_hbm.at[pl.ds(bs, KE)], dst2.at[slot])
        pltpu.async_copy(x_hbm.at[src2.at[slot]], rows2.at[slot], sems[slot])

      def drain(slot):
        pltpu.make_async_copy(
            x_hbm.at[src2.at[slot]], rows2.at[slot], sems[slot]).wait()
        pltpu.sync_copy(rows2.at[slot], acc.at[dst2.at[slot]], add=True)

      @pl.when(nblk > 0)
      def _():
        issue(0, 0)

      def pbody(p, _):
        b1 = 2 * p + 1
        b2 = 2 * p + 2

        @pl.when(b1 < nblk)
        def _():
          issue(b1, 1)

        drain(0)

        @pl.when(b2 < nblk)
        def _():
          issue(b2, 0)

        @pl.when(b1 < nblk)
        def _():
          drain(1)

        return 0

      lax.fori_loop(0, (nblk + 1) // 2, pbody, 0)
      plsc.subcore_barrier()

      for s7 in range(5):
        lrow = sub * 560 + s7 * WROWS
        grow = cid * CHUNK + lrow
        pltpu.sync_copy(acc.at[pl.ds(lrow, WROWS)], wb_v)
        pltpu.sync_copy(wb_v, out_hbm.at[pl.ds(grow, WROWS)])
      plsc.subcore_barrier()

  return k(h, src_arr, dstl_arr, a_arr)


def _mlp_body(epsb_ref, x_ref, agg_ref, w1_ref, b1_ref, w2_ref, b2_ref,
              y_ref, st_ref):
  i = pl.program_id(0)
  h0 = epsb_ref[...] * x_ref[...] + agg_ref[...]
  z = jnp.maximum(
      jnp.dot(h0, w1_ref[...], preferred_element_type=jnp.float32) + b1_ref[...],
      0.0)
  y = jnp.maximum(
      jnp.dot(z, w2_ref[...], preferred_element_type=jnp.float32) + b2_ref[...],
      0.0)
  y_ref[...] = y

  @pl.when(i == 0)
  def _():
    st_ref[...] = jnp.zeros_like(st_ref)

  st_ref[0:1, :] += jnp.sum(y, axis=0, keepdims=True)
  st_ref[1:2, :] += jnp.sum(y * y, axis=0, keepdims=True)


def _mlp(epsb, x, agg, w1, b1, w2, b2, nblocks):
  return pl.pallas_call(
      _mlp_body,
      grid=(nblocks,),
      in_specs=[
          pl.BlockSpec((1, 128), lambda i: (0, 0)),
          pl.BlockSpec((RBLK, 128), lambda i: (i, 0)),
          pl.BlockSpec((RBLK, 128), lambda i: (i, 0)),
          pl.BlockSpec((128, 128), lambda i: (0, 0)),
          pl.BlockSpec((1, 128), lambda i: (0, 0)),
          pl.BlockSpec((128, 128), lambda i: (0, 0)),
          pl.BlockSpec((1, 128), lambda i: (0, 0)),
      ],
      out_specs=[
          pl.BlockSpec((RBLK, 128), lambda i: (i, 0)),
          pl.BlockSpec((8, 128), lambda i: (0, 0)),
      ],
      out_shape=[
          jax.ShapeDtypeStruct((nblocks * RBLK, 128), jnp.float32),
          jax.ShapeDtypeStruct((8, 128), jnp.float32),
      ],
  )(epsb, x, agg, w1, b1, w2, b2)


def _bn_body(n_ref, st_ref, g_ref, be_ref, y_ref, o_ref):
  n = n_ref[0, 0]
  mean = st_ref[0:1, :] / n
  var = st_ref[1:2, :] / n - mean * mean
  scale = g_ref[...] * lax.rsqrt(var + 1e-5)
  o_ref[...] = (y_ref[...] - mean) * scale + be_ref[...]


def _bn(nval, st, g, be, y, nblocks):
  return pl.pallas_call(
      _bn_body,
      grid=(nblocks,),
      in_specs=[
          pl.BlockSpec((1, 1), lambda i: (0, 0)),
          pl.BlockSpec((8, 128), lambda i: (0, 0)),
          pl.BlockSpec((1, 128), lambda i: (0, 0)),
          pl.BlockSpec((1, 128), lambda i: (0, 0)),
          pl.BlockSpec((RBLK, 128), lambda i: (i, 0)),
      ],
      out_specs=pl.BlockSpec((RBLK, 128), lambda i: (i, 0)),
      out_shape=jax.ShapeDtypeStruct((nblocks * RBLK, 128), jnp.float32),
  )(nval, st, g, be, y)


def _bnpool_body(n_ref, st_ref, g_ref, be_ref, y_ref, batch_ref,
                 sums_ref, cnts_ref):
  i = pl.program_id(0)
  n = n_ref[0, 0]
  mean = st_ref[0:1, :] / n
  var = st_ref[1:2, :] / n - mean * mean
  scale = g_ref[...] * lax.rsqrt(var + 1e-5)
  ybn = (y_ref[...] - mean) * scale + be_ref[...]
  b = batch_ref[0, 0, :]
  oh = (lax.broadcasted_iota(jnp.int32, (RBLK, 128), 1) == b[:, None]
        ).astype(jnp.float32)

  @pl.when(i == 0)
  def _():
    sums_ref[...] = jnp.zeros_like(sums_ref)
    cnts_ref[...] = jnp.zeros_like(cnts_ref)

  sums_ref[...] += lax.dot_general(
      oh, ybn, (((0,), (0,)), ((), ())), preferred_element_type=jnp.float32)
  cnts_ref[...] += lax.dot_general(
      oh, jnp.ones((RBLK, 128), jnp.float32), (((0,), (0,)), ((), ())),
      preferred_element_type=jnp.float32)


def _bnpool(nval, st, g, be, y, batch3, nblocks):
  return pl.pallas_call(
      _bnpool_body,
      grid=(nblocks,),
      in_specs=[
          pl.BlockSpec((1, 1), lambda i: (0, 0)),
          pl.BlockSpec((8, 128), lambda i: (0, 0)),
          pl.BlockSpec((1, 128), lambda i: (0, 0)),
          pl.BlockSpec((1, 128), lambda i: (0, 0)),
          pl.BlockSpec((RBLK, 128), lambda i: (i, 0)),
          pl.BlockSpec((1, 1, RBLK), lambda i: (i, 0, 0)),
      ],
      out_specs=[
          pl.BlockSpec((128, 128), lambda i: (0, 0)),
          pl.BlockSpec((128, 128), lambda i: (0, 0)),
      ],
      out_shape=[
          jax.ShapeDtypeStruct((128, 128), jnp.float32),
          jax.ShapeDtypeStruct((128, 128), jnp.float32),
      ],
  )(nval, st, g, be, y, batch3)


def _head_body(sums_ref, cnts_ref, w1_ref, b1_ref, w2_ref, b2_ref, o_ref):
  xr = sums_ref[...] / jnp.maximum(cnts_ref[...], 1.0)
  o = jnp.maximum(
      jnp.dot(xr, w1_ref[...], preferred_element_type=jnp.float32) + b1_ref[...],
      0.0)
  o2 = jnp.dot(o, w2_ref[...], preferred_element_type=jnp.float32) + b2_ref[...]
  m = jnp.max(o2, axis=1, keepdims=True)
  e = jnp.exp(o2 - m)
  s = jnp.sum(e, axis=1, keepdims=True)
  o_ref[...] = (o2 - m) - jnp.log(s)


def _head(sums, cnts, w1, b1, w2, b2, C):
  return pl.pallas_call(
      _head_body,
      out_shape=jax.ShapeDtypeStruct((128, C), jnp.float32),
  )(sums, cnts, w1, b1, w2, b2)


def kernel(x, edge_index, batch, eps1, W1a, b1a, W1b, b1b, g1, be1,
           eps2, W2a, b2a, W2b, b2b, g2, be2,
           eps3, W3a, b3a, W3b, b3b, g3, be3,
           lin1_W, lin1_b, lin2_W, lin2_b):
  N, FIN = x.shape
  H = W1a.shape[1]
  C = lin2_W.shape[1]
  E = edge_index.shape[1]
  nblocks = N // RBLK
  E_pad = E + NCHUNK * TILE_BLK

  src = edge_index[0]
  dst = edge_index[1]
  perm = jnp.argsort(dst)
  src_s = src[perm]
  dst_s = dst[perm]
  bounds = jnp.searchsorted(
      dst_s, jnp.arange(0, NPAD + 1, CHUNK, dtype=jnp.int32)).astype(jnp.int32)
  cnts = jnp.diff(bounds)
  plen = ((cnts + TILE_BLK - 1) // TILE_BLK) * TILE_BLK
  poff = jnp.concatenate(
      [jnp.zeros((1,), jnp.int32), jnp.cumsum(plen)]).astype(jnp.int32)
  p = jnp.arange(E_pad, dtype=jnp.int32)
  seg = jnp.clip(jnp.searchsorted(poff, p, side="right").astype(jnp.int32) - 1,
                 0, NCHUNK - 1)
  l = p - poff[seg]
  valid = l < cnts[seg]
  j = jnp.where(valid, bounds[seg] + l, 0)
  src_arr = jnp.where(valid, src_s[j], 0).astype(jnp.int32)
  dstl_arr = jnp.where(valid, dst_s[j] - seg * CHUNK, CHUNK).astype(jnp.int32)
  a_arr = jnp.concatenate(
      [poff[:NCHUNK], plen // TILE_BLK,
       jnp.zeros((16 - 2 * NCHUNK,), jnp.int32)]).astype(jnp.int32)

  xp = jnp.pad(x, ((0, 0), (0, H - FIN)))
  W1a_p = jnp.pad(W1a, ((0, H - FIN), (0, 0)))
  nv = jnp.full((1, 1), float(N), jnp.float32)

  def layer(h, epsv, wa, ba, wb, bb):
    agg = _segsum_sc(h, src_arr, dstl_arr, a_arr)
    epsb = jnp.full((1, 128), 1.0, jnp.float32) + epsv
    y, st = _mlp(epsb, h, agg, wa, ba.reshape(1, -1), wb, bb.reshape(1, -1),
                 nblocks)
    return y, st

  y1, st1 = layer(xp, eps1, W1a_p, b1a, W1b, b1b)
  h1 = _bn(nv, st1, g1.reshape(1, -1), be1.reshape(1, -1), y1, nblocks)
  y2, st2 = layer(h1, eps2, W2a, b2a, W2b, b2b)
  h2 = _bn(nv, st2, g2.reshape(1, -1), be2.reshape(1, -1), y2, nblocks)
  y3, st3 = layer(h2, eps3, W3a, b3a, W3b, b3b)
  batch3 = batch.reshape(nblocks, 1, RBLK).astype(jnp.int32)
  sums, cnts_pool = _bnpool(nv, st3, g3.reshape(1, -1), be3.reshape(1, -1),
                            y3, batch3, nblocks)
  return _head(sums, cnts_pool, lin1_W, lin1_b.reshape(1, -1),
               lin2_W, lin2_b.reshape(1, -1), C)


# speedup vs baseline: 4.0648x; 1.0049x over previous
import functools

import jax
import jax.numpy as jnp
from jax import lax
from jax.experimental import pallas as pl
from jax.experimental.pallas import tpu as pltpu
from jax.experimental.pallas import tpu_sc as plsc

NC = 2
NS = 16
KE = 128
TILE_BLK = NS * KE
CHUNK = 8960
NCHUNK = 6
NPAD = NCHUNK * CHUNK
ACC_R = 9120
WROWS = 112
RBLK = 2000


def _segsum_sc(h, src_arr, dstl_arr, a_arr):
  mesh = plsc.VectorSubcoreMesh(
      core_axis_name="c", subcore_axis_name="s", num_cores=NC, num_subcores=NS)

  @functools.partial(
      pl.kernel,
      out_type=jax.ShapeDtypeStruct((NPAD, 128), jnp.float32),
      mesh=mesh,
      scratch_types=[
          pltpu.VMEM((16,), jnp.int32),
          pltpu.VMEM((2, KE), jnp.int32),
          pltpu.VMEM((2, KE), jnp.int32),
          pltpu.VMEM((2, KE, 128), jnp.float32),
          pltpu.VMEM((WROWS, 128), jnp.float32),
          pltpu.VMEM_SHARED((ACC_R, 128), jnp.float32),
          pltpu.SemaphoreType.DMA,
          pltpu.SemaphoreType.DMA,
      ],
  )
  def k(x_hbm, src_hbm, dstl_hbm, a_hbm, out_hbm,
        a_v, src2, dst2, rows2, wb_v, acc, sem0, sem1):
    core = lax.axis_index("c")
    sub = lax.axis_index("s")
    pltpu.sync_copy(a_hbm, a_v)
    av = a_v[...]

    def geta(i):
      return jnp.squeeze(lax.slice(av, (i,), (i + 1,)))

    zvec = jnp.zeros((16,), jnp.float32)

    def zbody(t, _):
      zb = t // 8
      c8 = t % 8
      wb_v[zb, pl.ds(c8 * 16, 16)] = zvec
      return 0

    for kk in range(NCHUNK // 2):
      cid = core + 2 * kk
      off = jnp.where(core == 0, geta(2 * kk), geta(2 * kk + 1))
      nblk = jnp.where(core == 0, geta(NCHUNK + 2 * kk),
                       geta(NCHUNK + 2 * kk + 1))
      pcnt = nblk * KE

      lax.fori_loop(0, WROWS * 8, zbody, 0)
      for s6 in range(6):
        pltpu.sync_copy(wb_v, acc.at[pl.ds(sub * 561 + s6 * WROWS, WROWS)])
      plsc.subcore_barrier()

      tstart = pl.multiple_of(off + sub * pcnt, KE)
      sems = (sem0, sem1)

      def issue(b, slot):
        bs = pl.multiple_of(tstart + b * KE, KE)
        pltpu.sync_copy(src_hbm.at[pl.ds(bs, KE)], src2.at[slot])
        pltpu.sync_copy(dstl_hbm.at[pl.ds(bs, KE)], dst2.at[slot])
        pltpu.async_copy(x_hbm.at[src2.at[slot]], rows2.at[slot], sems[slot])

      def drain(slot):
        pltpu.make_async_copy(
            x_hbm.at[src2.at[slot]], rows2.at[slot], sems[slot]).wait()
        pltpu.sync_copy(rows2.at[slot], acc.at[dst2.at[slot]], add=True)

      @pl.when(nblk > 0)
      def _():
        issue(0, 0)

      def pbody(p, _):
        b1 = 2 * p + 1
        b2 = 2 * p + 2

        @pl.when(b1 < nblk)
        def _():
          issue(b1, 1)

        drain(0)

        @pl.when(b2 < nblk)
        def _():
          issue(b2, 0)

        @pl.when(b1 < nblk)
        def _():
          drain(1)

        return 0

      lax.fori_loop(0, (nblk + 1) // 2, pbody, 0)
      plsc.subcore_barrier()

      for s7 in range(5):
        lrow = sub * 560 + s7 * WROWS
        grow = cid * CHUNK + lrow
        pltpu.sync_copy(acc.at[pl.ds(lrow, WROWS)], wb_v)
        pltpu.sync_copy(wb_v, out_hbm.at[pl.ds(grow, WROWS)])
      plsc.subcore_barrier()

  return k(h, src_arr, dstl_arr, a_arr)


def _mlp_body(epsb_ref, x_ref, agg_ref, w1_ref, b1_ref, w2_ref, b2_ref,
              y_ref, st_ref):
  i = pl.program_id(0)
  h0 = epsb_ref[...] * x_ref[...] + agg_ref[...]
  z = jnp.maximum(
      jnp.dot(h0, w1_ref[...], preferred_element_type=jnp.float32) + b1_ref[...],
      0.0)
  y = jnp.maximum(
      jnp.dot(z, w2_ref[...], preferred_element_type=jnp.float32) + b2_ref[...],
      0.0)
  y_ref[...] = y

  @pl.when(i == 0)
  def _():
    st_ref[...] = jnp.zeros_like(st_ref)

  st_ref[0:1, :] += jnp.sum(y, axis=0, keepdims=True)
  st_ref[1:2, :] += jnp.sum(y * y, axis=0, keepdims=True)


def _mlp(epsb, x, agg, w1, b1, w2, b2, nblocks):
  return pl.pallas_call(
      _mlp_body,
      grid=(nblocks,),
      in_specs=[
          pl.BlockSpec((1, 128), lambda i: (0, 0)),
          pl.BlockSpec((RBLK, 128), lambda i: (i, 0)),
          pl.BlockSpec((RBLK, 128), lambda i: (i, 0)),
          pl.BlockSpec((128, 128), lambda i: (0, 0)),
          pl.BlockSpec((1, 128), lambda i: (0, 0)),
          pl.BlockSpec((128, 128), lambda i: (0, 0)),
          pl.BlockSpec((1, 128), lambda i: (0, 0)),
      ],
      out_specs=[
          pl.BlockSpec((RBLK, 128), lambda i: (i, 0)),
          pl.BlockSpec((8, 128), lambda i: (0, 0)),
      ],
      out_shape=[
          jax.ShapeDtypeStruct((nblocks * RBLK, 128), jnp.float32),
          jax.ShapeDtypeStruct((8, 128), jnp.float32),
      ],
  )(epsb, x, agg, w1, b1, w2, b2)


def _bn_body(n_ref, st_ref, g_ref, be_ref, y_ref, o_ref):
  n = n_ref[0, 0]
  mean = st_ref[0:1, :] / n
  var = st_ref[1:2, :] / n - mean * mean
  scale = g_ref[...] * lax.rsqrt(var + 1e-5)
  o_ref[...] = (y_ref[...] - mean) * scale + be_ref[...]


def _bn(nval, st, g, be, y, nblocks):
  return pl.pallas_call(
      _bn_body,
      grid=(nblocks,),
      in_specs=[
          pl.BlockSpec((1, 1), lambda i: (0, 0)),
          pl.BlockSpec((8, 128), lambda i: (0, 0)),
          pl.BlockSpec((1, 128), lambda i: (0, 0)),
          pl.BlockSpec((1, 128), lambda i: (0, 0)),
          pl.BlockSpec((RBLK, 128), lambda i: (i, 0)),
      ],
      out_specs=pl.BlockSpec((RBLK, 128), lambda i: (i, 0)),
      out_shape=jax.ShapeDtypeStruct((nblocks * RBLK, 128), jnp.float32),
  )(nval, st, g, be, y)


def _bnpool_body(n_ref, st_ref, g_ref, be_ref, y_ref, batch_ref,
                 sums_ref, cnts_ref):
  i = pl.program_id(0)
  n = n_ref[0, 0]
  mean = st_ref[0:1, :] / n
  var = st_ref[1:2, :] / n - mean * mean
  scale = g_ref[...] * lax.rsqrt(var + 1e-5)
  ybn = (y_ref[...] - mean) * scale + be_ref[...]
  b = batch_ref[0, 0, :]
  oh = (lax.broadcasted_iota(jnp.int32, (RBLK, 128), 1) == b[:, None]
        ).astype(jnp.float32)

  @pl.when(i == 0)
  def _():
    sums_ref[...] = jnp.zeros_like(sums_ref)
    cnts_ref[...] = jnp.zeros_like(cnts_ref)

  sums_ref[...] += lax.dot_general(
      oh, ybn, (((0,), (0,)), ((), ())), preferred_element_type=jnp.float32)
  cnts_ref[...] += lax.dot_general(
      oh, jnp.ones((RBLK, 128), jnp.float32), (((0,), (0,)), ((), ())),
      preferred_element_type=jnp.float32)


def _bnpool(nval, st, g, be, y, batch3, nblocks):
  return pl.pallas_call(
      _bnpool_body,
      grid=(nblocks,),
      in_specs=[
          pl.BlockSpec((1, 1), lambda i: (0, 0)),
          pl.BlockSpec((8, 128), lambda i: (0, 0)),
          pl.BlockSpec((1, 128), lambda i: (0, 0)),
          pl.BlockSpec((1, 128), lambda i: (0, 0)),
          pl.BlockSpec((RBLK, 128), lambda i: (i, 0)),
          pl.BlockSpec((1, 1, RBLK), lambda i: (i, 0, 0)),
      ],
      out_specs=[
          pl.BlockSpec((128, 128), lambda i: (0, 0)),
          pl.BlockSpec((128, 128), lambda i: (0, 0)),
      ],
      out_shape=[
          jax.ShapeDtypeStruct((128, 128), jnp.float32),
          jax.ShapeDtypeStruct((128, 128), jnp.float32),
      ],
  )(nval, st, g, be, y, batch3)


def _head_body(sums_ref, cnts_ref, w1_ref, b1_ref, w2_ref, b2_ref, o_ref):
  xr = sums_ref[...] / jnp.maximum(cnts_ref[...], 1.0)
  o = jnp.maximum(
      jnp.dot(xr, w1_ref[...], preferred_element_type=jnp.float32) + b1_ref[...],
      0.0)
  o2 = jnp.dot(o, w2_ref[...], preferred_element_type=jnp.float32) + b2_ref[...]
  m = jnp.max(o2, axis=1, keepdims=True)
  e = jnp.exp(o2 - m)
  s = jnp.sum(e, axis=1, keepdims=True)
  o_ref[...] = (o2 - m) - jnp.log(s)


def _head(sums, cnts, w1, b1, w2, b2, C):
  return pl.pallas_call(
      _head_body,
      out_shape=jax.ShapeDtypeStruct((128, C), jnp.float32),
  )(sums, cnts, w1, b1, w2, b2)


def kernel(x, edge_index, batch, eps1, W1a, b1a, W1b, b1b, g1, be1,
           eps2, W2a, b2a, W2b, b2b, g2, be2,
           eps3, W3a, b3a, W3b, b3b, g3, be3,
           lin1_W, lin1_b, lin2_W, lin2_b):
  N, FIN = x.shape
  H = W1a.shape[1]
  C = lin2_W.shape[1]
  E = edge_index.shape[1]
  nblocks = N // RBLK
  E_pad = E + NCHUNK * TILE_BLK

  src = edge_index[0]
  dst = edge_index[1]
  ch = dst // CHUNK
  packed = (ch.astype(jnp.uint32) << 20) | jnp.arange(E, dtype=jnp.uint32)
  sp = jnp.sort(packed)
  perm = (sp & jnp.uint32(0xFFFFF)).astype(jnp.int32)
  bounds = jnp.searchsorted(
      sp, (jnp.arange(NCHUNK + 1, dtype=jnp.uint32) << 20)).astype(jnp.int32)
  cnts = jnp.diff(bounds)
  plen = ((cnts + TILE_BLK - 1) // TILE_BLK) * TILE_BLK
  poff = jnp.concatenate(
      [jnp.zeros((1,), jnp.int32), jnp.cumsum(plen)]).astype(jnp.int32)
  p = jnp.arange(E_pad, dtype=jnp.int32)
  seg = jnp.clip(jnp.searchsorted(poff, p, side="right").astype(jnp.int32) - 1,
                 0, NCHUNK - 1)
  l = p - poff[seg]
  valid = l < cnts[seg]
  perm2 = perm[jnp.where(valid, bounds[seg] + l, 0)]
  src_arr = jnp.where(valid, src[perm2], 0).astype(jnp.int32)
  dstl_arr = jnp.where(valid, dst[perm2] - seg * CHUNK, CHUNK).astype(jnp.int32)
  a_arr = jnp.concatenate(
      [poff[:NCHUNK], plen // TILE_BLK,
       jnp.zeros((16 - 2 * NCHUNK,), jnp.int32)]).astype(jnp.int32)

  xp = jnp.pad(x, ((0, 0), (0, H - FIN)))
  W1a_p = jnp.pad(W1a, ((0, H - FIN), (0, 0)))
  nv = jnp.full((1, 1), float(N), jnp.float32)

  def layer(h, epsv, wa, ba, wb, bb):
    agg = _segsum_sc(h, src_arr, dstl_arr, a_arr)
    epsb = jnp.full((1, 128), 1.0, jnp.float32) + epsv
    y, st = _mlp(epsb, h, agg, wa, ba.reshape(1, -1), wb, bb.reshape(1, -1),
                 nblocks)
    return y, st

  y1, st1 = layer(xp, eps1, W1a_p, b1a, W1b, b1b)
  h1 = _bn(nv, st1, g1.reshape(1, -1), be1.reshape(1, -1), y1, nblocks)
  y2, st2 = layer(h1, eps2, W2a, b2a, W2b, b2b)
  h2 = _bn(nv, st2, g2.reshape(1, -1), be2.reshape(1, -1), y2, nblocks)
  y3, st3 = layer(h2, eps3, W3a, b3a, W3b, b3b)
  batch3 = batch.reshape(nblocks, 1, RBLK).astype(jnp.int32)
  sums, cnts_pool = _bnpool(nv, st3, g3.reshape(1, -1), be3.reshape(1, -1),
                            y3, batch3, nblocks)
  return _head(sums, cnts_pool, lin1_W, lin1_b.reshape(1, -1),
               lin2_W, lin2_b.reshape(1, -1), C)
